# Initial kernel scaffold; baseline (speedup 1.0000x reference)
import jax
import jax.numpy as jnp
from jax.experimental import pallas as pl


def kernel(xyz, color, params):
    raise NotImplementedError("write your pallas kernel here")


# speedup vs baseline: 9.2973x; 9.2973x over previous
import functools

import jax
import jax.numpy as jnp
from jax import lax
from jax.experimental import pallas as pl
from jax.experimental.pallas import tpu as pltpu
from jax.experimental.pallas import tpu_sc as plsc

B, N, OUT = 8, 2048, 128
N1, N2, K = 1024, 205, 64
EPS = 1e-5
NEG_INF = float("-inf")


def _fps_body(pos_ref, q_ref, *, n, n_samples):
    px = pos_ref[0, 0:1, :]
    py = pos_ref[0, 1:2, :]
    pz = pos_ref[0, 2:3, :]
    lane = lax.broadcasted_iota(jnp.int32, (1, n), 1)
    lane_s = lax.broadcasted_iota(jnp.int32, (1, n_samples), 1)

    lx = px[0:1, 0:1]
    ly = py[0:1, 0:1]
    lz = pz[0:1, 0:1]
    qx0 = jnp.where(lane_s == 0, lx, 0.0)
    qy0 = jnp.where(lane_s == 0, ly, 0.0)
    qz0 = jnp.where(lane_s == 0, lz, 0.0)
    dists0 = jnp.full((1, n), jnp.inf, dtype=jnp.float32)

    def body(i, state):
        dists, qx, qy, qz, lx, ly, lz = state
        dx = px - lx
        dy = py - ly
        dz = pz - lz
        d = (dx * dx + dy * dy) + dz * dz
        dists = jnp.minimum(dists, d)
        m = jnp.max(dists, axis=1, keepdims=True)
        sel = jnp.min(jnp.where(dists == m, lane, n), axis=1, keepdims=True)
        selmask = lane == sel
        nlx = jnp.sum(jnp.where(selmask, px, 0.0), axis=1, keepdims=True)
        nly = jnp.sum(jnp.where(selmask, py, 0.0), axis=1, keepdims=True)
        nlz = jnp.sum(jnp.where(selmask, pz, 0.0), axis=1, keepdims=True)
        at_i = lane_s == i
        qx = jnp.where(at_i, nlx, qx)
        qy = jnp.where(at_i, nly, qy)
        qz = jnp.where(at_i, nlz, qz)
        return dists, qx, qy, qz, nlx, nly, nlz

    _, qx, qy, qz, _, _, _ = lax.fori_loop(
        1, n_samples, body, (dists0, qx0, qy0, qz0, lx, ly, lz)
    )
    q_ref[0, 0:1, :] = qx
    q_ref[0, 1:2, :] = qy
    q_ref[0, 2:3, :] = qz


def _fps(pos_rows, n, n_samples):
    return pl.pallas_call(
        functools.partial(_fps_body, n=n, n_samples=n_samples),
        grid=(B,),
        in_specs=[pl.BlockSpec((1, 3, n), lambda b: (b, 0, 0))],
        out_specs=pl.BlockSpec((1, 3, n_samples), lambda b: (b, 0, 0)),
        out_shape=jax.ShapeDtypeStruct((B, 3, n_samples), jnp.float32),
        compiler_params=pltpu.CompilerParams(dimension_semantics=("parallel",)),
    )(pos_rows)


def _knn_body(pos_cols_ref, q_rows_ref, idx_ref, cnt_ref, work_ref, *, n, q, r2s):
    pxc = pos_cols_ref[0, :, 0:1]
    pyc = pos_cols_ref[0, :, 1:2]
    pzc = pos_cols_ref[0, :, 2:3]
    qx = q_rows_ref[0, 0:1, :]
    qy = q_rows_ref[0, 1:2, :]
    qz = q_rows_ref[0, 2:3, :]
    dx = qx - pxc
    dy = qy - pyc
    dz = qz - pzc
    work_ref[...] = (dx * dx + dy * dy) + dz * dz
    sub = lax.broadcasted_iota(jnp.int32, (n, q), 0)
    ksub = lax.broadcasted_iota(jnp.int32, (K, q), 0)

    def body(k, state):
        idxc, c0, c1 = state
        work = work_ref[...]
        m = jnp.min(work, axis=0, keepdims=True)
        mi = jnp.min(jnp.where(work == m, sub, n), axis=0, keepdims=True)
        idxc = jnp.where(ksub == k, mi, idxc)
        c0 = c0 + jnp.where(m <= r2s[0], 1, 0)
        c1 = c1 + jnp.where(m <= r2s[1], 1, 0)
        work_ref[...] = jnp.where(sub == mi, jnp.inf, work)
        return idxc, c0, c1

    idxc0 = jnp.zeros((K, q), jnp.int32)
    z = jnp.zeros((1, q), jnp.int32)
    idxc, c0, c1 = lax.fori_loop(0, K, body, (idxc0, z, z))
    idx_ref[0, :, :] = idxc
    cnt_ref[0, 0:1, :] = c0
    cnt_ref[0, 1:2, :] = c1


def _knn(pos_cols, q_rows, n, q, r2s):
    nr = len(r2s)
    return pl.pallas_call(
        functools.partial(_knn_body, n=n, q=q, r2s=r2s),
        grid=(B,),
        in_specs=[
            pl.BlockSpec((1, n, 3), lambda b: (b, 0, 0)),
            pl.BlockSpec((1, 3, q), lambda b: (b, 0, 0)),
        ],
        out_specs=[
            pl.BlockSpec((1, K, q), lambda b: (b, 0, 0)),
            pl.BlockSpec((1, nr, q), lambda b: (b, 0, 0)),
        ],
        out_shape=[
            jax.ShapeDtypeStruct((B, K, q), jnp.int32),
            jax.ShapeDtypeStruct((B, nr, q), jnp.int32),
        ],
        scratch_shapes=[pltpu.VMEM((n, q), jnp.float32)],
        compiler_params=pltpu.CompilerParams(dimension_semantics=("parallel",)),
    )(pos_cols, q_rows)


def _a1_body(pos_cols_ref, col_cols_ref, w_ref, b_ref, a_ref, *, n, f):
    acc = jnp.broadcast_to(b_ref[0:1, :], (n, f))
    for c in range(3):
        acc = acc + col_cols_ref[0, :, c : c + 1] * w_ref[c : c + 1, :]
    for c in range(3):
        acc = acc + pos_cols_ref[0, :, c : c + 1] * w_ref[3 + c : 4 + c, :]
    a_ref[0, :, :] = acc


def _a1(pos_cols, color_cols, w1, b1):
    f = w1.shape[1]
    return pl.pallas_call(
        functools.partial(_a1_body, n=N, f=f),
        grid=(B,),
        in_specs=[
            pl.BlockSpec((1, N, 3), lambda b: (b, 0, 0)),
            pl.BlockSpec((1, N, 3), lambda b: (b, 0, 0)),
            pl.BlockSpec((6, f), lambda b: (0, 0)),
            pl.BlockSpec((1, f), lambda b: (0, 0)),
        ],
        out_specs=pl.BlockSpec((1, N, f), lambda b: (b, 0, 0)),
        out_shape=jax.ShapeDtypeStruct((B, N, f), jnp.float32),
        compiler_params=pltpu.CompilerParams(dimension_semantics=("parallel",)),
    )(pos_cols, color_cols, w1, b1)


def _a2_body(x_ref, pos_cols_ref, wx_ref, wp_ref, b_ref, a_ref, *, n, f):
    acc = jnp.dot(x_ref[0], wx_ref[...], preferred_element_type=jnp.float32)
    acc = acc + b_ref[0:1, :]
    for c in range(3):
        acc = acc + pos_cols_ref[0, :, c : c + 1] * wp_ref[c : c + 1, :]
    a_ref[0, :, :] = acc


def _a2(x1, pos_cols, wx, wp, b):
    n, fin = x1.shape[1], x1.shape[2]
    f = wx.shape[1]
    return pl.pallas_call(
        functools.partial(_a2_body, n=n, f=f),
        grid=(B,),
        in_specs=[
            pl.BlockSpec((1, n, fin), lambda b: (b, 0, 0)),
            pl.BlockSpec((1, n, 3), lambda b: (b, 0, 0)),
            pl.BlockSpec((fin, f), lambda b: (0, 0)),
            pl.BlockSpec((3, f), lambda b: (0, 0)),
            pl.BlockSpec((1, f), lambda b: (0, 0)),
        ],
        out_specs=pl.BlockSpec((1, n, f), lambda b: (b, 0, 0)),
        out_shape=jax.ShapeDtypeStruct((B, n, f), jnp.float32),
        compiler_params=pltpu.CompilerParams(dimension_semantics=("parallel",)),
    )(x1, pos_cols, wx, wp, b)


def _sc_gather(table, idx, chunk):
    info = plsc.get_sparse_core_info()
    nc, ns = info.num_cores, info.num_subcores
    nw = nc * ns
    rows, d = idx.shape[0], table.shape[1]
    b_per_w = rows // nw
    n_chunks = b_per_w // chunk
    mesh = plsc.VectorSubcoreMesh(core_axis_name="c", subcore_axis_name="s")

    @functools.partial(
        pl.kernel,
        mesh=mesh,
        out_type=jax.ShapeDtypeStruct((rows, d), jnp.float32),
        scratch_types=[
            pltpu.VMEM((chunk,), jnp.int32),
            pltpu.VMEM((chunk, d), jnp.float32),
            pltpu.SemaphoreType.DMA,
        ],
    )
    def k(table_hbm, idx_hbm, out_hbm, idx_v, rows_v, sem):
        wid = lax.axis_index("s") * nc + lax.axis_index("c")
        base = wid * b_per_w

        @pl.loop(0, n_chunks)
        def _chunk(c):
            off = base + c * chunk
            pltpu.sync_copy(idx_hbm.at[pl.ds(off, chunk)], idx_v)
            pltpu.async_copy(table_hbm.at[idx_v], rows_v, sem).wait()
            pltpu.sync_copy(rows_v, out_hbm.at[pl.ds(off, chunk)])

    return k(table, idx)


def _conv_body(ag_ref, q_cols_ref, cnt_cols_ref, w1p_ref, s1_ref, t1_ref,
               w2_ref, b2_ref, s2_ref, t2_ref, out_ref, *, q, f1, f2, gw):
    cq = q_cols_ref[0, :, 0:1] * w1p_ref[0:1, :]
    cq = cq + q_cols_ref[0, :, 1:2] * w1p_ref[1:2, :]
    cq = cq + q_cols_ref[0, :, 2:3] * w1p_ref[2:3, :]
    cnt0 = cnt_cols_ref[0, :, 0:1]
    cnt1 = cnt_cols_ref[0, :, 1:2]

    def body(k, state):
        acc0, acc1 = state
        h1 = jax.nn.relu(ag_ref[0, k][:, 0:f1] - cq)
        h1 = h1 * s1_ref[0:1, :] + t1_ref[0:1, :]
        h2 = jnp.dot(h1, w2_ref[...], preferred_element_type=jnp.float32)
        h2 = jax.nn.relu(h2 + b2_ref[0:1, :])
        h2 = h2 * s2_ref[0:1, :] + t2_ref[0:1, :]
        acc0 = jnp.where(k < cnt0, jnp.maximum(acc0, h2), acc0)
        acc1 = jnp.where(k < cnt1, jnp.maximum(acc1, h2), acc1)
        return acc0, acc1

    neg = jnp.full((q, f2), NEG_INF, dtype=jnp.float32)
    acc0, acc1 = lax.fori_loop(0, K, body, (neg, neg))
    out_ref[0, :, 0:f2] = jnp.where(acc0 > NEG_INF, acc0, 0.0)
    out_ref[0, :, f2 : 2 * f2] = jnp.where(acc1 > NEG_INF, acc1, 0.0)


def _conv(ag, q_cols, cnt_cols, w1p, s1, t1, w2, b2, s2, t2, q, qb=None):
    f1, f2 = w2.shape
    gw = ag.shape[3]
    qb = q if qb is None else qb
    return pl.pallas_call(
        functools.partial(_conv_body, q=qb, f1=f1, f2=f2, gw=gw),
        grid=(B, q // qb),
        in_specs=[
            pl.BlockSpec((1, K, qb, gw), lambda b, i: (b, 0, i, 0)),
            pl.BlockSpec((1, qb, 3), lambda b, i: (b, i, 0)),
            pl.BlockSpec((1, qb, 2), lambda b, i: (b, i, 0)),
            pl.BlockSpec((3, f1), lambda b, i: (0, 0)),
            pl.BlockSpec((1, f1), lambda b, i: (0, 0)),
            pl.BlockSpec((1, f1), lambda b, i: (0, 0)),
            pl.BlockSpec((f1, f2), lambda b, i: (0, 0)),
            pl.BlockSpec((1, f2), lambda b, i: (0, 0)),
            pl.BlockSpec((1, f2), lambda b, i: (0, 0)),
            pl.BlockSpec((1, f2), lambda b, i: (0, 0)),
        ],
        out_specs=pl.BlockSpec((1, qb, 2 * f2), lambda b, i: (b, i, 0)),
        out_shape=jax.ShapeDtypeStruct((B, q, 2 * f2), jnp.float32),
        compiler_params=pltpu.CompilerParams(
            dimension_semantics=("parallel", "arbitrary")
        ),
    )(ag, q_cols, cnt_cols, w1p, s1, t1, w2, b2, s2, t2)


def _head_a_body(x_ref, pos_cols_ref, wx_ref, wp_ref, b_ref, s_ref, t_ref,
                 out_ref, *, n, f):
    h = jnp.dot(x_ref[0], wx_ref[...], preferred_element_type=jnp.float32)
    h = h + b_ref[0:1, :]
    for c in range(3):
        h = h + pos_cols_ref[0, :, c : c + 1] * wp_ref[c : c + 1, :]
    h = jax.nn.relu(h)
    h = h * s_ref[0:1, :] + t_ref[0:1, :]
    out_ref[0, :, :] = jnp.max(h, axis=0, keepdims=True)


def _head_a(x2, pos_cols, wx, wp, b, s, t):
    n, fin = x2.shape[1], x2.shape[2]
    f = wx.shape[1]
    return pl.pallas_call(
        functools.partial(_head_a_body, n=n, f=f),
        grid=(B,),
        in_specs=[
            pl.BlockSpec((1, n, fin), lambda b: (b, 0, 0)),
            pl.BlockSpec((1, n, 3), lambda b: (b, 0, 0)),
            pl.BlockSpec((fin, f), lambda b: (0, 0)),
            pl.BlockSpec((3, f), lambda b: (0, 0)),
            pl.BlockSpec((1, f), lambda b: (0, 0)),
            pl.BlockSpec((1, f), lambda b: (0, 0)),
            pl.BlockSpec((1, f), lambda b: (0, 0)),
        ],
        out_specs=pl.BlockSpec((1, 1, f), lambda b: (b, 0, 0)),
        out_shape=jax.ShapeDtypeStruct((B, 1, f), jnp.float32),
        compiler_params=pltpu.CompilerParams(dimension_semantics=("parallel",)),
    )(x2, pos_cols, wx, wp, b, s, t)


def _head_b_body(x_ref, w1_ref, b1_ref, w2_ref, b2_ref, out_ref):
    h = jnp.dot(x_ref[0], w1_ref[...], preferred_element_type=jnp.float32)
    h = jax.nn.relu(h + b1_ref[0:1, :])
    o = jnp.dot(h, w2_ref[...], preferred_element_type=jnp.float32)
    o = o + b2_ref[0:1, :]
    nrm = jnp.sqrt(jnp.sum(o * o, axis=1, keepdims=True))
    out_ref[0, :, :] = o / nrm


def _head_b(x3, w1, b1, w2, b2):
    f1, f2 = w1.shape[1], w2.shape[1]
    return pl.pallas_call(
        _head_b_body,
        grid=(1,),
        in_specs=[
            pl.BlockSpec((1, B, x3.shape[2]), lambda i: (0, 0, 0)),
            pl.BlockSpec(w1.shape, lambda i: (0, 0)),
            pl.BlockSpec((1, f1), lambda i: (0, 0)),
            pl.BlockSpec(w2.shape, lambda i: (0, 0)),
            pl.BlockSpec((1, f2), lambda i: (0, 0)),
        ],
        out_specs=pl.BlockSpec((1, B, f2), lambda i: (0, 0, 0)),
        out_shape=jax.ShapeDtypeStruct((1, B, f2), jnp.float32),
    )(x3, w1, b1, w2, b2)


def _bn_fold(layer):
    w, b, g, be = layer
    s = g / jnp.sqrt(1.0 + EPS)
    return w, b.reshape(1, -1), s.reshape(1, -1), be.reshape(1, -1)


def kernel(xyz, color, params):
    sa1 = [_bn_fold(l) for l in params["sa1"]]
    sa2 = [_bn_fold(l) for l in params["sa2"]]
    sa3 = [_bn_fold(l) for l in params["sa3"]]
    w1_1, b1_1, s1_1, t1_1 = sa1[0]
    w2_1, b2_1, s2_1, t2_1 = sa1[1]
    w1_2, b1_2, s1_2, t1_2 = sa2[0]
    w2_2, b2_2, s2_2, t2_2 = sa2[1]
    w3, b3, s3, t3 = sa3[0]
    wl1, bl1 = params["lin1"]
    wl2, bl2 = params["lin2"]

    pos_rows = jnp.swapaxes(xyz, 1, 2)
    pos_cols = xyz
    color_cols = color

    q1_rows = _fps(pos_rows, N, N1)
    q1_cols = jnp.swapaxes(q1_rows, 1, 2)
    nbr1, cnt1 = _knn(pos_cols, q1_rows, N, N1, (0.2 * 0.2, 0.1 * 0.1))
    a1 = _a1(pos_cols, color_cols, w1_1, b1_1)
    a1p = jnp.pad(a1.reshape(B * N, 64), ((0, 0), (0, 64)))
    offs1 = (jnp.arange(B, dtype=jnp.int32) * N).reshape(B, 1, 1)
    idx1 = (nbr1 + offs1).reshape(B * K * N1)
    g1 = _sc_gather(a1p, idx1, 512)
    g1 = g1.reshape(B, K, N1, 128)
    cnt1_cols = jnp.swapaxes(cnt1, 1, 2)
    x1 = _conv(g1, q1_cols, cnt1_cols, w1_1[3:6], s1_1, t1_1,
               w2_1, b2_1, s2_1, t2_1, N1, qb=256)

    q2_rows = _fps(q1_rows, N1, N2)
    q2_cols = jnp.swapaxes(q2_rows, 1, 2)
    nbr2, cnt2 = _knn(q1_cols, q2_rows, N1, N2, (0.35 * 0.35, 0.5 * 0.5))
    a2 = _a2(x1, q1_cols, w1_2[:128], w1_2[128:131], b1_2)
    offs2 = (jnp.arange(B, dtype=jnp.int32) * N1).reshape(B, 1, 1)
    idx2 = (nbr2 + offs2).reshape(B * K * N2)
    g2 = _sc_gather(a2.reshape(B * N1, 128), idx2, 656)
    g2 = g2.reshape(B, K, N2, 128)
    cnt2_cols = jnp.swapaxes(cnt2, 1, 2)
    x2 = _conv(g2, q2_cols, cnt2_cols, w1_2[128:131], s1_2, t1_2,
               w2_2, b2_2, s2_2, t2_2, N2)

    x3 = _head_a(x2, q2_cols, w3[:512], w3[512:515], b3, s3, t3)
    x3 = x3.reshape(1, B, 1024)
    out = _head_b(x3, wl1, bl1.reshape(1, -1), wl2, bl2.reshape(1, -1))
    return out.reshape(B, OUT)


# speedup vs baseline: 23.0283x; 2.4769x over previous
import functools

import jax
import jax.numpy as jnp
from jax import lax
from jax.experimental import pallas as pl
from jax.experimental.pallas import tpu as pltpu
from jax.experimental.pallas import tpu_sc as plsc

B, N, OUT = 8, 2048, 128
N1, N2, K = 1024, 205, 64
EPS = 1e-5
NEG_INF = float("-inf")


def _fps_body(pos_ref, q_ref, *, n, n_samples):
    px = pos_ref[0]
    py = pos_ref[1]
    pz = pos_ref[2]
    lane = lax.broadcasted_iota(jnp.int32, (B, n), 1)
    lane_s = lax.broadcasted_iota(jnp.int32, (B, n_samples), 1)

    lx = px[:, 0:1]
    ly = py[:, 0:1]
    lz = pz[:, 0:1]
    at0 = lane_s == 0
    qx0 = jnp.where(at0, lx, 0.0)
    qy0 = jnp.where(at0, ly, 0.0)
    qz0 = jnp.where(at0, lz, 0.0)
    dists0 = jnp.full((B, n), jnp.inf, dtype=jnp.float32)

    def body(i, state):
        dists, qx, qy, qz, lx, ly, lz = state
        dx = px - lx
        dy = py - ly
        dz = pz - lz
        d = (dx * dx + dy * dy) + dz * dz
        dists = jnp.minimum(dists, d)
        m = jnp.max(dists, axis=1, keepdims=True)
        sel = jnp.min(jnp.where(dists == m, lane, n), axis=1, keepdims=True)
        selmask = lane == sel
        nlx = jnp.sum(jnp.where(selmask, px, 0.0), axis=1, keepdims=True)
        nly = jnp.sum(jnp.where(selmask, py, 0.0), axis=1, keepdims=True)
        nlz = jnp.sum(jnp.where(selmask, pz, 0.0), axis=1, keepdims=True)
        at_i = lane_s == i
        qx = jnp.where(at_i, nlx, qx)
        qy = jnp.where(at_i, nly, qy)
        qz = jnp.where(at_i, nlz, qz)
        return dists, qx, qy, qz, nlx, nly, nlz

    _, qx, qy, qz, _, _, _ = lax.fori_loop(
        1, n_samples, body, (dists0, qx0, qy0, qz0, lx, ly, lz)
    )
    q_ref[0] = qx
    q_ref[1] = qy
    q_ref[2] = qz


def _fps(pos_coord, n, n_samples):
    return pl.pallas_call(
        functools.partial(_fps_body, n=n, n_samples=n_samples),
        grid=(1,),
        in_specs=[pl.BlockSpec((3, B, n), lambda i: (0, 0, 0))],
        out_specs=pl.BlockSpec((3, B, n_samples), lambda i: (0, 0, 0)),
        out_shape=jax.ShapeDtypeStruct((3, B, n_samples), jnp.float32),
    )(pos_coord)


def _knn_body(pos_cols_ref, q_rows_ref, idx_ref, cnt_ref, work_ref, *, n, q, r2s):
    pxc = pos_cols_ref[0, :, 0:1]
    pyc = pos_cols_ref[0, :, 1:2]
    pzc = pos_cols_ref[0, :, 2:3]
    qx = q_rows_ref[0, 0:1, :]
    qy = q_rows_ref[0, 1:2, :]
    qz = q_rows_ref[0, 2:3, :]
    dx = qx - pxc
    dy = qy - pyc
    dz = qz - pzc
    work_ref[...] = (dx * dx + dy * dy) + dz * dz
    sub = lax.broadcasted_iota(jnp.int32, (n, q), 0)
    ksub = lax.broadcasted_iota(jnp.int32, (K, q), 0)

    def body(k, state):
        idxc, c0, c1 = state
        work = work_ref[...]
        m = jnp.min(work, axis=0, keepdims=True)
        mi = jnp.min(jnp.where(work == m, sub, n), axis=0, keepdims=True)
        idxc = jnp.where(ksub == k, mi, idxc)
        c0 = c0 + jnp.where(m <= r2s[0], 1, 0)
        c1 = c1 + jnp.where(m <= r2s[1], 1, 0)
        work_ref[...] = jnp.where(sub == mi, jnp.inf, work)
        return idxc, c0, c1

    idxc0 = jnp.zeros((K, q), jnp.int32)
    z = jnp.zeros((1, q), jnp.int32)
    idxc, c0, c1 = lax.fori_loop(0, K, body, (idxc0, z, z))
    idx_ref[0, :, :] = idxc
    cnt_ref[0, 0:1, :] = c0
    cnt_ref[0, 1:2, :] = c1


def _knn(pos_cols, q_rows, n, q, r2s):
    nr = len(r2s)
    return pl.pallas_call(
        functools.partial(_knn_body, n=n, q=q, r2s=r2s),
        grid=(B,),
        in_specs=[
            pl.BlockSpec((1, n, 3), lambda b: (b, 0, 0)),
            pl.BlockSpec((1, 3, q), lambda b: (b, 0, 0)),
        ],
        out_specs=[
            pl.BlockSpec((1, K, q), lambda b: (b, 0, 0)),
            pl.BlockSpec((1, nr, q), lambda b: (b, 0, 0)),
        ],
        out_shape=[
            jax.ShapeDtypeStruct((B, K, q), jnp.int32),
            jax.ShapeDtypeStruct((B, nr, q), jnp.int32),
        ],
        scratch_shapes=[pltpu.VMEM((n, q), jnp.float32)],
        compiler_params=pltpu.CompilerParams(dimension_semantics=("parallel",)),
    )(pos_cols, q_rows)


def _a1_body(pos_cols_ref, col_cols_ref, w_ref, b_ref, a_ref, *, n, f):
    acc = jnp.broadcast_to(b_ref[0:1, :], (n, f))
    for c in range(3):
        acc = acc + col_cols_ref[0, :, c : c + 1] * w_ref[c : c + 1, :]
    for c in range(3):
        acc = acc + pos_cols_ref[0, :, c : c + 1] * w_ref[3 + c : 4 + c, :]
    a_ref[0, :, :] = acc


def _a1(pos_cols, color_cols, w1, b1):
    f = w1.shape[1]
    return pl.pallas_call(
        functools.partial(_a1_body, n=N, f=f),
        grid=(B,),
        in_specs=[
            pl.BlockSpec((1, N, 3), lambda b: (b, 0, 0)),
            pl.BlockSpec((1, N, 3), lambda b: (b, 0, 0)),
            pl.BlockSpec((6, f), lambda b: (0, 0)),
            pl.BlockSpec((1, f), lambda b: (0, 0)),
        ],
        out_specs=pl.BlockSpec((1, N, f), lambda b: (b, 0, 0)),
        out_shape=jax.ShapeDtypeStruct((B, N, f), jnp.float32),
        compiler_params=pltpu.CompilerParams(dimension_semantics=("parallel",)),
    )(pos_cols, color_cols, w1, b1)


def _a2_body(x_ref, pos_cols_ref, wx_ref, wp_ref, b_ref, a_ref, *, n, f):
    acc = jnp.dot(x_ref[0], wx_ref[...], preferred_element_type=jnp.float32)
    acc = acc + b_ref[0:1, :]
    for c in range(3):
        acc = acc + pos_cols_ref[0, :, c : c + 1] * wp_ref[c : c + 1, :]
    a_ref[0, :, :] = acc


def _a2(x1, pos_cols, wx, wp, b):
    n, fin = x1.shape[1], x1.shape[2]
    f = wx.shape[1]
    return pl.pallas_call(
        functools.partial(_a2_body, n=n, f=f),
        grid=(B,),
        in_specs=[
            pl.BlockSpec((1, n, fin), lambda b: (b, 0, 0)),
            pl.BlockSpec((1, n, 3), lambda b: (b, 0, 0)),
            pl.BlockSpec((fin, f), lambda b: (0, 0)),
            pl.BlockSpec((3, f), lambda b: (0, 0)),
            pl.BlockSpec((1, f), lambda b: (0, 0)),
        ],
        out_specs=pl.BlockSpec((1, n, f), lambda b: (b, 0, 0)),
        out_shape=jax.ShapeDtypeStruct((B, n, f), jnp.float32),
        compiler_params=pltpu.CompilerParams(dimension_semantics=("parallel",)),
    )(x1, pos_cols, wx, wp, b)


def _sc_gather(table, idx, chunk):
    info = plsc.get_sparse_core_info()
    nc, ns = info.num_cores, info.num_subcores
    nw = nc * ns
    rows, d = idx.shape[0], table.shape[1]
    b_per_w = rows // nw
    n_chunks = b_per_w // chunk
    mesh = plsc.VectorSubcoreMesh(core_axis_name="c", subcore_axis_name="s")

    @functools.partial(
        pl.kernel,
        mesh=mesh,
        out_type=jax.ShapeDtypeStruct((rows, d), jnp.float32),
        scratch_types=[
            pltpu.VMEM((chunk,), jnp.int32),
            pltpu.VMEM((chunk, d), jnp.float32),
            pltpu.SemaphoreType.DMA,
        ],
    )
    def k(table_hbm, idx_hbm, out_hbm, idx_v, rows_v, sem):
        wid = lax.axis_index("s") * nc + lax.axis_index("c")
        base = wid * b_per_w

        @pl.loop(0, n_chunks)
        def _chunk(c):
            off = base + c * chunk
            pltpu.sync_copy(idx_hbm.at[pl.ds(off, chunk)], idx_v)
            pltpu.async_copy(table_hbm.at[idx_v], rows_v, sem).wait()
            pltpu.sync_copy(rows_v, out_hbm.at[pl.ds(off, chunk)])

    return k(table, idx)


def _conv_body(ag_ref, q_cols_ref, cnt_cols_ref, w1p_ref, s1_ref, t1_ref,
               w2_ref, b2_ref, s2_ref, t2_ref, out_ref, *, q, f1, f2, gw):
    cq = q_cols_ref[0, :, 0:1] * w1p_ref[0:1, :]
    cq = cq + q_cols_ref[0, :, 1:2] * w1p_ref[1:2, :]
    cq = cq + q_cols_ref[0, :, 2:3] * w1p_ref[2:3, :]
    cnt0 = cnt_cols_ref[0, :, 0:1]
    cnt1 = cnt_cols_ref[0, :, 1:2]

    w2 = w2_ref[...]

    def half(k, acc0, acc1):
        h1 = jax.nn.relu(ag_ref[0, k][:, 0:f1] - cq)
        h1 = h1 * s1_ref[0:1, :] + t1_ref[0:1, :]
        h2 = jnp.dot(h1, w2, preferred_element_type=jnp.float32)
        h2 = jax.nn.relu(h2 + b2_ref[0:1, :])
        h2 = h2 * s2_ref[0:1, :] + t2_ref[0:1, :]
        acc0 = jnp.where(k < cnt0, jnp.maximum(acc0, h2), acc0)
        acc1 = jnp.where(k < cnt1, jnp.maximum(acc1, h2), acc1)
        return acc0, acc1

    def body(k, state):
        a0a, a1a, a0b, a1b = state
        a0a, a1a = half(k, a0a, a1a)
        a0b, a1b = half(k + K // 2, a0b, a1b)
        return a0a, a1a, a0b, a1b

    neg = jnp.full((q, f2), NEG_INF, dtype=jnp.float32)
    a0a, a1a, a0b, a1b = lax.fori_loop(0, K // 2, body, (neg, neg, neg, neg))
    acc0 = jnp.maximum(a0a, a0b)
    acc1 = jnp.maximum(a1a, a1b)
    out_ref[0, :, 0:f2] = jnp.where(acc0 > NEG_INF, acc0, 0.0)
    out_ref[0, :, f2 : 2 * f2] = jnp.where(acc1 > NEG_INF, acc1, 0.0)


def _conv(ag, q_cols, cnt_cols, w1p, s1, t1, w2, b2, s2, t2, q, qb=None):
    f1, f2 = w2.shape
    gw = ag.shape[3]
    qb = q if qb is None else qb
    return pl.pallas_call(
        functools.partial(_conv_body, q=qb, f1=f1, f2=f2, gw=gw),
        grid=(B, q // qb),
        in_specs=[
            pl.BlockSpec((1, K, qb, gw), lambda b, i: (b, 0, i, 0)),
            pl.BlockSpec((1, qb, 3), lambda b, i: (b, i, 0)),
            pl.BlockSpec((1, qb, 2), lambda b, i: (b, i, 0)),
            pl.BlockSpec((3, f1), lambda b, i: (0, 0)),
            pl.BlockSpec((1, f1), lambda b, i: (0, 0)),
            pl.BlockSpec((1, f1), lambda b, i: (0, 0)),
            pl.BlockSpec((f1, f2), lambda b, i: (0, 0)),
            pl.BlockSpec((1, f2), lambda b, i: (0, 0)),
            pl.BlockSpec((1, f2), lambda b, i: (0, 0)),
            pl.BlockSpec((1, f2), lambda b, i: (0, 0)),
        ],
        out_specs=pl.BlockSpec((1, qb, 2 * f2), lambda b, i: (b, i, 0)),
        out_shape=jax.ShapeDtypeStruct((B, q, 2 * f2), jnp.float32),
        compiler_params=pltpu.CompilerParams(
            dimension_semantics=("parallel", "arbitrary")
        ),
    )(ag, q_cols, cnt_cols, w1p, s1, t1, w2, b2, s2, t2)


def _head_a_body(x_ref, pos_cols_ref, wx_ref, wp_ref, b_ref, s_ref, t_ref,
                 out_ref, *, n, f):
    h = jnp.dot(x_ref[0], wx_ref[...], preferred_element_type=jnp.float32)
    h = h + b_ref[0:1, :]
    for c in range(3):
        h = h + pos_cols_ref[0, :, c : c + 1] * wp_ref[c : c + 1, :]
    h = jax.nn.relu(h)
    h = h * s_ref[0:1, :] + t_ref[0:1, :]
    out_ref[0, :, :] = jnp.max(h, axis=0, keepdims=True)


def _head_a(x2, pos_cols, wx, wp, b, s, t):
    n, fin = x2.shape[1], x2.shape[2]
    f = wx.shape[1]
    return pl.pallas_call(
        functools.partial(_head_a_body, n=n, f=f),
        grid=(B,),
        in_specs=[
            pl.BlockSpec((1, n, fin), lambda b: (b, 0, 0)),
            pl.BlockSpec((1, n, 3), lambda b: (b, 0, 0)),
            pl.BlockSpec((fin, f), lambda b: (0, 0)),
            pl.BlockSpec((3, f), lambda b: (0, 0)),
            pl.BlockSpec((1, f), lambda b: (0, 0)),
            pl.BlockSpec((1, f), lambda b: (0, 0)),
            pl.BlockSpec((1, f), lambda b: (0, 0)),
        ],
        out_specs=pl.BlockSpec((1, 1, f), lambda b: (b, 0, 0)),
        out_shape=jax.ShapeDtypeStruct((B, 1, f), jnp.float32),
        compiler_params=pltpu.CompilerParams(dimension_semantics=("parallel",)),
    )(x2, pos_cols, wx, wp, b, s, t)


def _head_b_body(x_ref, w1_ref, b1_ref, w2_ref, b2_ref, out_ref):
    h = jnp.dot(x_ref[0], w1_ref[...], preferred_element_type=jnp.float32)
    h = jax.nn.relu(h + b1_ref[0:1, :])
    o = jnp.dot(h, w2_ref[...], preferred_element_type=jnp.float32)
    o = o + b2_ref[0:1, :]
    nrm = jnp.sqrt(jnp.sum(o * o, axis=1, keepdims=True))
    out_ref[0, :, :] = o / nrm


def _head_b(x3, w1, b1, w2, b2):
    f1, f2 = w1.shape[1], w2.shape[1]
    return pl.pallas_call(
        _head_b_body,
        grid=(1,),
        in_specs=[
            pl.BlockSpec((1, B, x3.shape[2]), lambda i: (0, 0, 0)),
            pl.BlockSpec(w1.shape, lambda i: (0, 0)),
            pl.BlockSpec((1, f1), lambda i: (0, 0)),
            pl.BlockSpec(w2.shape, lambda i: (0, 0)),
            pl.BlockSpec((1, f2), lambda i: (0, 0)),
        ],
        out_specs=pl.BlockSpec((1, B, f2), lambda i: (0, 0, 0)),
        out_shape=jax.ShapeDtypeStruct((1, B, f2), jnp.float32),
    )(x3, w1, b1, w2, b2)


def _bn_fold(layer):
    w, b, g, be = layer
    s = g / jnp.sqrt(1.0 + EPS)
    return w, b.reshape(1, -1), s.reshape(1, -1), be.reshape(1, -1)


def kernel(xyz, color, params):
    sa1 = [_bn_fold(l) for l in params["sa1"]]
    sa2 = [_bn_fold(l) for l in params["sa2"]]
    sa3 = [_bn_fold(l) for l in params["sa3"]]
    w1_1, b1_1, s1_1, t1_1 = sa1[0]
    w2_1, b2_1, s2_1, t2_1 = sa1[1]
    w1_2, b1_2, s1_2, t1_2 = sa2[0]
    w2_2, b2_2, s2_2, t2_2 = sa2[1]
    w3, b3, s3, t3 = sa3[0]
    wl1, bl1 = params["lin1"]
    wl2, bl2 = params["lin2"]

    pos_coord = jnp.transpose(xyz, (2, 0, 1))
    pos_cols = xyz
    color_cols = color

    q1_coord = _fps(pos_coord, N, N1)
    q1_rows = jnp.swapaxes(q1_coord, 0, 1)
    q1_cols = jnp.transpose(q1_coord, (1, 2, 0))
    nbr1, cnt1 = _knn(pos_cols, q1_rows, N, N1, (0.2 * 0.2, 0.1 * 0.1))
    a1 = _a1(pos_cols, color_cols, w1_1, b1_1)
    a1p = jnp.pad(a1.reshape(B * N, 64), ((0, 0), (0, 64)))
    offs1 = (jnp.arange(B, dtype=jnp.int32) * N).reshape(B, 1, 1)
    idx1 = (nbr1 + offs1).reshape(B * K * N1)
    g1 = _sc_gather(a1p, idx1, 512)
    g1 = g1.reshape(B, K, N1, 128)
    cnt1_cols = jnp.swapaxes(cnt1, 1, 2)
    x1 = _conv(g1, q1_cols, cnt1_cols, w1_1[3:6], s1_1, t1_1,
               w2_1, b2_1, s2_1, t2_1, N1, qb=256)

    q2_coord = _fps(q1_coord, N1, N2)
    q2_rows = jnp.swapaxes(q2_coord, 0, 1)
    q2_cols = jnp.transpose(q2_coord, (1, 2, 0))
    nbr2, cnt2 = _knn(q1_cols, q2_rows, N1, N2, (0.35 * 0.35, 0.5 * 0.5))
    a2 = _a2(x1, q1_cols, w1_2[:128], w1_2[128:131], b1_2)
    offs2 = (jnp.arange(B, dtype=jnp.int32) * N1).reshape(B, 1, 1)
    idx2 = (nbr2 + offs2).reshape(B * K * N2)
    g2 = _sc_gather(a2.reshape(B * N1, 128), idx2, 656)
    g2 = g2.reshape(B, K, N2, 128)
    cnt2_cols = jnp.swapaxes(cnt2, 1, 2)
    x2 = _conv(g2, q2_cols, cnt2_cols, w1_2[128:131], s1_2, t1_2,
               w2_2, b2_2, s2_2, t2_2, N2)

    x3 = _head_a(x2, q2_cols, w3[:512], w3[512:515], b3, s3, t3)
    x3 = x3.reshape(1, B, 1024)
    out = _head_b(x3, wl1, bl1.reshape(1, -1), wl2, bl2.reshape(1, -1))
    return out.reshape(B, OUT)


# speedup vs baseline: 23.6504x; 1.0270x over previous
import functools

import jax
import jax.numpy as jnp
from jax import lax
from jax.experimental import pallas as pl
from jax.experimental.pallas import tpu as pltpu
from jax.experimental.pallas import tpu_sc as plsc

B, N, OUT = 8, 2048, 128
N1, N2, K = 1024, 205, 64
EPS = 1e-5
NEG_INF = float("-inf")


def _fps_body(pos_ref, q_ref, *, n, n_samples):
    px = pos_ref[0]
    py = pos_ref[1]
    pz = pos_ref[2]
    lane = lax.broadcasted_iota(jnp.int32, (B, n), 1)
    lane_s = lax.broadcasted_iota(jnp.int32, (B, n_samples), 1)

    lx = px[:, 0:1]
    ly = py[:, 0:1]
    lz = pz[:, 0:1]
    at0 = lane_s == 0
    qx0 = jnp.where(at0, lx, 0.0)
    qy0 = jnp.where(at0, ly, 0.0)
    qz0 = jnp.where(at0, lz, 0.0)
    dists0 = jnp.full((B, n), jnp.inf, dtype=jnp.float32)

    hn = n // 2
    laneL = lane[:, 0:hn]
    laneR = lane[:, hn:n]

    def body(i, state):
        dists, qx, qy, qz, lx, ly, lz = state
        dx = px - lx
        dy = py - ly
        dz = pz - lz
        d = (dx * dx + dy * dy) + dz * dz
        dists = jnp.minimum(dists, d)
        dL = dists[:, 0:hn]
        dR = dists[:, hn:n]
        m = jnp.maximum(
            jnp.max(dL, axis=1, keepdims=True),
            jnp.max(dR, axis=1, keepdims=True),
        )
        sel = jnp.minimum(
            jnp.min(jnp.where(dL == m, laneL, n), axis=1, keepdims=True),
            jnp.min(jnp.where(dR == m, laneR, n), axis=1, keepdims=True),
        )
        mskL = laneL == sel
        mskR = laneR == sel
        nlx = jnp.sum(jnp.where(mskL, px[:, 0:hn], 0.0), axis=1, keepdims=True) + \
              jnp.sum(jnp.where(mskR, px[:, hn:n], 0.0), axis=1, keepdims=True)
        nly = jnp.sum(jnp.where(mskL, py[:, 0:hn], 0.0), axis=1, keepdims=True) + \
              jnp.sum(jnp.where(mskR, py[:, hn:n], 0.0), axis=1, keepdims=True)
        nlz = jnp.sum(jnp.where(mskL, pz[:, 0:hn], 0.0), axis=1, keepdims=True) + \
              jnp.sum(jnp.where(mskR, pz[:, hn:n], 0.0), axis=1, keepdims=True)
        at_i = lane_s == i
        qx = jnp.where(at_i, nlx, qx)
        qy = jnp.where(at_i, nly, qy)
        qz = jnp.where(at_i, nlz, qz)
        return dists, qx, qy, qz, nlx, nly, nlz

    _, qx, qy, qz, _, _, _ = lax.fori_loop(
        1, n_samples, body, (dists0, qx0, qy0, qz0, lx, ly, lz)
    )
    q_ref[0] = qx
    q_ref[1] = qy
    q_ref[2] = qz


def _fps(pos_coord, n, n_samples):
    return pl.pallas_call(
        functools.partial(_fps_body, n=n, n_samples=n_samples),
        grid=(1,),
        in_specs=[pl.BlockSpec((3, B, n), lambda i: (0, 0, 0))],
        out_specs=pl.BlockSpec((3, B, n_samples), lambda i: (0, 0, 0)),
        out_shape=jax.ShapeDtypeStruct((3, B, n_samples), jnp.float32),
    )(pos_coord)


def _knn_body(pos_cols_ref, q_rows_ref, idx_ref, cnt_ref, work_ref, *, n, q, r2s):
    pxc = pos_cols_ref[0, :, 0:1]
    pyc = pos_cols_ref[0, :, 1:2]
    pzc = pos_cols_ref[0, :, 2:3]
    qx = q_rows_ref[0, 0:1, :]
    qy = q_rows_ref[0, 1:2, :]
    qz = q_rows_ref[0, 2:3, :]
    dx = qx - pxc
    dy = qy - pyc
    dz = qz - pzc
    work_ref[...] = (dx * dx + dy * dy) + dz * dz
    h = n // 2
    subL = lax.broadcasted_iota(jnp.int32, (h, q), 0)
    subR = subL + h
    ksub = lax.broadcasted_iota(jnp.int32, (K, q), 0)

    def body(k, state):
        idxc, c0, c1 = state
        wL = work_ref[0:h, :]
        wR = work_ref[h:n, :]
        mL = jnp.min(wL, axis=0, keepdims=True)
        mR = jnp.min(wR, axis=0, keepdims=True)
        m = jnp.minimum(mL, mR)
        miL = jnp.min(jnp.where(wL == m, subL, n), axis=0, keepdims=True)
        miR = jnp.min(jnp.where(wR == m, subR, n), axis=0, keepdims=True)
        mi = jnp.minimum(miL, miR)
        idxc = jnp.where(ksub == k, mi, idxc)
        c0 = c0 + jnp.where(m <= r2s[0], 1, 0)
        c1 = c1 + jnp.where(m <= r2s[1], 1, 0)
        work_ref[0:h, :] = jnp.where(subL == mi, jnp.inf, wL)
        work_ref[h:n, :] = jnp.where(subR == mi, jnp.inf, wR)
        return idxc, c0, c1

    idxc0 = jnp.zeros((K, q), jnp.int32)
    z = jnp.zeros((1, q), jnp.int32)
    idxc, c0, c1 = lax.fori_loop(0, K, body, (idxc0, z, z))
    idx_ref[0, :, :] = idxc
    cnt_ref[0, 0:1, :] = c0
    cnt_ref[0, 1:2, :] = c1


def _knn(pos_cols, q_rows, n, q, r2s):
    nr = len(r2s)
    return pl.pallas_call(
        functools.partial(_knn_body, n=n, q=q, r2s=r2s),
        grid=(B,),
        in_specs=[
            pl.BlockSpec((1, n, 3), lambda b: (b, 0, 0)),
            pl.BlockSpec((1, 3, q), lambda b: (b, 0, 0)),
        ],
        out_specs=[
            pl.BlockSpec((1, K, q), lambda b: (b, 0, 0)),
            pl.BlockSpec((1, nr, q), lambda b: (b, 0, 0)),
        ],
        out_shape=[
            jax.ShapeDtypeStruct((B, K, q), jnp.int32),
            jax.ShapeDtypeStruct((B, nr, q), jnp.int32),
        ],
        scratch_shapes=[pltpu.VMEM((n, q), jnp.float32)],
        compiler_params=pltpu.CompilerParams(dimension_semantics=("parallel",)),
    )(pos_cols, q_rows)


def _a1_body(pos_cols_ref, col_cols_ref, w_ref, b_ref, a_ref, *, n, f):
    acc = jnp.broadcast_to(b_ref[0:1, :], (n, f))
    for c in range(3):
        acc = acc + col_cols_ref[0, :, c : c + 1] * w_ref[c : c + 1, :]
    for c in range(3):
        acc = acc + pos_cols_ref[0, :, c : c + 1] * w_ref[3 + c : 4 + c, :]
    a_ref[0, :, :] = acc


def _a1(pos_cols, color_cols, w1, b1):
    f = w1.shape[1]
    return pl.pallas_call(
        functools.partial(_a1_body, n=N, f=f),
        grid=(B,),
        in_specs=[
            pl.BlockSpec((1, N, 3), lambda b: (b, 0, 0)),
            pl.BlockSpec((1, N, 3), lambda b: (b, 0, 0)),
            pl.BlockSpec((6, f), lambda b: (0, 0)),
            pl.BlockSpec((1, f), lambda b: (0, 0)),
        ],
        out_specs=pl.BlockSpec((1, N, f), lambda b: (b, 0, 0)),
        out_shape=jax.ShapeDtypeStruct((B, N, f), jnp.float32),
        compiler_params=pltpu.CompilerParams(dimension_semantics=("parallel",)),
    )(pos_cols, color_cols, w1, b1)


def _a2_body(x_ref, pos_cols_ref, wx_ref, wp_ref, b_ref, a_ref, *, n, f):
    acc = jnp.dot(x_ref[0], wx_ref[...], preferred_element_type=jnp.float32)
    acc = acc + b_ref[0:1, :]
    for c in range(3):
        acc = acc + pos_cols_ref[0, :, c : c + 1] * wp_ref[c : c + 1, :]
    a_ref[0, :, :] = acc


def _a2(x1, pos_cols, wx, wp, b):
    n, fin = x1.shape[1], x1.shape[2]
    f = wx.shape[1]
    return pl.pallas_call(
        functools.partial(_a2_body, n=n, f=f),
        grid=(B,),
        in_specs=[
            pl.BlockSpec((1, n, fin), lambda b: (b, 0, 0)),
            pl.BlockSpec((1, n, 3), lambda b: (b, 0, 0)),
            pl.BlockSpec((fin, f), lambda b: (0, 0)),
            pl.BlockSpec((3, f), lambda b: (0, 0)),
            pl.BlockSpec((1, f), lambda b: (0, 0)),
        ],
        out_specs=pl.BlockSpec((1, n, f), lambda b: (b, 0, 0)),
        out_shape=jax.ShapeDtypeStruct((B, n, f), jnp.float32),
        compiler_params=pltpu.CompilerParams(dimension_semantics=("parallel",)),
    )(x1, pos_cols, wx, wp, b)


def _sc_gather(table, idx, chunk):
    info = plsc.get_sparse_core_info()
    nc, ns = info.num_cores, info.num_subcores
    nw = nc * ns
    rows, d = idx.shape[0], table.shape[1]
    b_per_w = rows // nw
    n_chunks = b_per_w // chunk
    mesh = plsc.VectorSubcoreMesh(core_axis_name="c", subcore_axis_name="s")

    @functools.partial(
        pl.kernel,
        mesh=mesh,
        out_type=jax.ShapeDtypeStruct((rows, d), jnp.float32),
        scratch_types=[
            pltpu.VMEM((chunk,), jnp.int32),
            pltpu.VMEM((chunk, d), jnp.float32),
            pltpu.SemaphoreType.DMA,
        ],
    )
    def k(table_hbm, idx_hbm, out_hbm, idx_v, rows_v, sem):
        wid = lax.axis_index("s") * nc + lax.axis_index("c")
        base = wid * b_per_w

        @pl.loop(0, n_chunks)
        def _chunk(c):
            off = base + c * chunk
            pltpu.sync_copy(idx_hbm.at[pl.ds(off, chunk)], idx_v)
            pltpu.async_copy(table_hbm.at[idx_v], rows_v, sem).wait()
            pltpu.sync_copy(rows_v, out_hbm.at[pl.ds(off, chunk)])

    return k(table, idx)


def _conv_body(ag_ref, q_cols_ref, cnt_cols_ref, w1p_ref, s1_ref, t1_ref,
               w2_ref, b2_ref, s2_ref, t2_ref, out_ref, *, q, f1, f2, gw):
    cq = q_cols_ref[0, :, 0:1] * w1p_ref[0:1, :]
    cq = cq + q_cols_ref[0, :, 1:2] * w1p_ref[1:2, :]
    cq = cq + q_cols_ref[0, :, 2:3] * w1p_ref[2:3, :]
    cnt0 = cnt_cols_ref[0, :, 0:1]
    cnt1 = cnt_cols_ref[0, :, 1:2]

    w2 = w2_ref[...]

    def half(k, acc0, acc1):
        h1 = jax.nn.relu(ag_ref[0, k][:, 0:f1] - cq)
        h1 = h1 * s1_ref[0:1, :] + t1_ref[0:1, :]
        h2 = jnp.dot(h1, w2, preferred_element_type=jnp.float32)
        h2 = jax.nn.relu(h2 + b2_ref[0:1, :])
        h2 = h2 * s2_ref[0:1, :] + t2_ref[0:1, :]
        acc0 = jnp.where(k < cnt0, jnp.maximum(acc0, h2), acc0)
        acc1 = jnp.where(k < cnt1, jnp.maximum(acc1, h2), acc1)
        return acc0, acc1

    def body(k, state):
        a0a, a1a, a0b, a1b = state
        a0a, a1a = half(k, a0a, a1a)
        a0b, a1b = half(k + K // 2, a0b, a1b)
        return a0a, a1a, a0b, a1b

    neg = jnp.full((q, f2), NEG_INF, dtype=jnp.float32)
    a0a, a1a, a0b, a1b = lax.fori_loop(0, K // 2, body, (neg, neg, neg, neg))
    acc0 = jnp.maximum(a0a, a0b)
    acc1 = jnp.maximum(a1a, a1b)
    out_ref[0, :, 0:f2] = jnp.where(acc0 > NEG_INF, acc0, 0.0)
    out_ref[0, :, f2 : 2 * f2] = jnp.where(acc1 > NEG_INF, acc1, 0.0)


def _conv(ag, q_cols, cnt_cols, w1p, s1, t1, w2, b2, s2, t2, q, qb=None):
    f1, f2 = w2.shape
    gw = ag.shape[3]
    qb = q if qb is None else qb
    return pl.pallas_call(
        functools.partial(_conv_body, q=qb, f1=f1, f2=f2, gw=gw),
        grid=(B, q // qb),
        in_specs=[
            pl.BlockSpec((1, K, qb, gw), lambda b, i: (b, 0, i, 0)),
            pl.BlockSpec((1, qb, 3), lambda b, i: (b, i, 0)),
            pl.BlockSpec((1, qb, 2), lambda b, i: (b, i, 0)),
            pl.BlockSpec((3, f1), lambda b, i: (0, 0)),
            pl.BlockSpec((1, f1), lambda b, i: (0, 0)),
            pl.BlockSpec((1, f1), lambda b, i: (0, 0)),
            pl.BlockSpec((f1, f2), lambda b, i: (0, 0)),
            pl.BlockSpec((1, f2), lambda b, i: (0, 0)),
            pl.BlockSpec((1, f2), lambda b, i: (0, 0)),
            pl.BlockSpec((1, f2), lambda b, i: (0, 0)),
        ],
        out_specs=pl.BlockSpec((1, qb, 2 * f2), lambda b, i: (b, i, 0)),
        out_shape=jax.ShapeDtypeStruct((B, q, 2 * f2), jnp.float32),
        compiler_params=pltpu.CompilerParams(
            dimension_semantics=("parallel", "arbitrary")
        ),
    )(ag, q_cols, cnt_cols, w1p, s1, t1, w2, b2, s2, t2)


def _head_a_body(x_ref, pos_cols_ref, wx_ref, wp_ref, b_ref, s_ref, t_ref,
                 out_ref, *, n, f):
    h = jnp.dot(x_ref[0], wx_ref[...], preferred_element_type=jnp.float32)
    h = h + b_ref[0:1, :]
    for c in range(3):
        h = h + pos_cols_ref[0, :, c : c + 1] * wp_ref[c : c + 1, :]
    h = jax.nn.relu(h)
    h = h * s_ref[0:1, :] + t_ref[0:1, :]
    out_ref[0, :, :] = jnp.max(h, axis=0, keepdims=True)


def _head_a(x2, pos_cols, wx, wp, b, s, t):
    n, fin = x2.shape[1], x2.shape[2]
    f = wx.shape[1]
    return pl.pallas_call(
        functools.partial(_head_a_body, n=n, f=f),
        grid=(B,),
        in_specs=[
            pl.BlockSpec((1, n, fin), lambda b: (b, 0, 0)),
            pl.BlockSpec((1, n, 3), lambda b: (b, 0, 0)),
            pl.BlockSpec((fin, f), lambda b: (0, 0)),
            pl.BlockSpec((3, f), lambda b: (0, 0)),
            pl.BlockSpec((1, f), lambda b: (0, 0)),
            pl.BlockSpec((1, f), lambda b: (0, 0)),
            pl.BlockSpec((1, f), lambda b: (0, 0)),
        ],
        out_specs=pl.BlockSpec((1, 1, f), lambda b: (b, 0, 0)),
        out_shape=jax.ShapeDtypeStruct((B, 1, f), jnp.float32),
        compiler_params=pltpu.CompilerParams(dimension_semantics=("parallel",)),
    )(x2, pos_cols, wx, wp, b, s, t)


def _head_b_body(x_ref, w1_ref, b1_ref, w2_ref, b2_ref, out_ref):
    h = jnp.dot(x_ref[0], w1_ref[...], preferred_element_type=jnp.float32)
    h = jax.nn.relu(h + b1_ref[0:1, :])
    o = jnp.dot(h, w2_ref[...], preferred_element_type=jnp.float32)
    o = o + b2_ref[0:1, :]
    nrm = jnp.sqrt(jnp.sum(o * o, axis=1, keepdims=True))
    out_ref[0, :, :] = o / nrm


def _head_b(x3, w1, b1, w2, b2):
    f1, f2 = w1.shape[1], w2.shape[1]
    return pl.pallas_call(
        _head_b_body,
        grid=(1,),
        in_specs=[
            pl.BlockSpec((1, B, x3.shape[2]), lambda i: (0, 0, 0)),
            pl.BlockSpec(w1.shape, lambda i: (0, 0)),
            pl.BlockSpec((1, f1), lambda i: (0, 0)),
            pl.BlockSpec(w2.shape, lambda i: (0, 0)),
            pl.BlockSpec((1, f2), lambda i: (0, 0)),
        ],
        out_specs=pl.BlockSpec((1, B, f2), lambda i: (0, 0, 0)),
        out_shape=jax.ShapeDtypeStruct((1, B, f2), jnp.float32),
    )(x3, w1, b1, w2, b2)


def _bn_fold(layer):
    w, b, g, be = layer
    s = g / jnp.sqrt(1.0 + EPS)
    return w, b.reshape(1, -1), s.reshape(1, -1), be.reshape(1, -1)


def kernel(xyz, color, params):
    sa1 = [_bn_fold(l) for l in params["sa1"]]
    sa2 = [_bn_fold(l) for l in params["sa2"]]
    sa3 = [_bn_fold(l) for l in params["sa3"]]
    w1_1, b1_1, s1_1, t1_1 = sa1[0]
    w2_1, b2_1, s2_1, t2_1 = sa1[1]
    w1_2, b1_2, s1_2, t1_2 = sa2[0]
    w2_2, b2_2, s2_2, t2_2 = sa2[1]
    w3, b3, s3, t3 = sa3[0]
    wl1, bl1 = params["lin1"]
    wl2, bl2 = params["lin2"]

    pos_coord = jnp.transpose(xyz, (2, 0, 1))
    pos_cols = xyz
    color_cols = color

    q1_coord = _fps(pos_coord, N, N1)
    q1_rows = jnp.swapaxes(q1_coord, 0, 1)
    q1_cols = jnp.transpose(q1_coord, (1, 2, 0))
    nbr1, cnt1 = _knn(pos_cols, q1_rows, N, N1, (0.2 * 0.2, 0.1 * 0.1))
    a1 = _a1(pos_cols, color_cols, w1_1, b1_1)
    a1p = jnp.pad(a1.reshape(B * N, 64), ((0, 0), (0, 64)))
    offs1 = (jnp.arange(B, dtype=jnp.int32) * N).reshape(B, 1, 1)
    idx1 = (nbr1 + offs1).reshape(B * K * N1)
    g1 = _sc_gather(a1p, idx1, 512)
    g1 = g1.reshape(B, K, N1, 128)
    cnt1_cols = jnp.swapaxes(cnt1, 1, 2)
    x1 = _conv(g1, q1_cols, cnt1_cols, w1_1[3:6], s1_1, t1_1,
               w2_1, b2_1, s2_1, t2_1, N1, qb=256)

    q2_coord = _fps(q1_coord, N1, N2)
    q2_rows = jnp.swapaxes(q2_coord, 0, 1)
    q2_cols = jnp.transpose(q2_coord, (1, 2, 0))
    nbr2, cnt2 = _knn(q1_cols, q2_rows, N1, N2, (0.35 * 0.35, 0.5 * 0.5))
    a2 = _a2(x1, q1_cols, w1_2[:128], w1_2[128:131], b1_2)
    offs2 = (jnp.arange(B, dtype=jnp.int32) * N1).reshape(B, 1, 1)
    idx2 = (nbr2 + offs2).reshape(B * K * N2)
    g2 = _sc_gather(a2.reshape(B * N1, 128), idx2, 656)
    g2 = g2.reshape(B, K, N2, 128)
    cnt2_cols = jnp.swapaxes(cnt2, 1, 2)
    x2 = _conv(g2, q2_cols, cnt2_cols, w1_2[128:131], s1_2, t1_2,
               w2_2, b2_2, s2_2, t2_2, N2)

    x3 = _head_a(x2, q2_cols, w3[:512], w3[512:515], b3, s3, t3)
    x3 = x3.reshape(1, B, 1024)
    out = _head_b(x3, wl1, bl1.reshape(1, -1), wl2, bl2.reshape(1, -1))
    return out.reshape(B, OUT)


# speedup vs baseline: 23.7823x; 1.0056x over previous
import functools

import jax
import jax.numpy as jnp
from jax import lax
from jax.experimental import pallas as pl
from jax.experimental.pallas import tpu as pltpu
from jax.experimental.pallas import tpu_sc as plsc

B, N, OUT = 8, 2048, 128
N1, N2, K = 1024, 205, 64
EPS = 1e-5
NEG_INF = float("-inf")


def _fps_body(pos_ref, q_ref, *, n, n_samples):
    px = pos_ref[0]
    py = pos_ref[1]
    pz = pos_ref[2]
    lane = lax.broadcasted_iota(jnp.int32, (B, n), 1)
    lane_s = lax.broadcasted_iota(jnp.int32, (B, n_samples), 1)

    lx = px[:, 0:1]
    ly = py[:, 0:1]
    lz = pz[:, 0:1]
    at0 = lane_s == 0
    qx0 = jnp.where(at0, lx, 0.0)
    qy0 = jnp.where(at0, ly, 0.0)
    qz0 = jnp.where(at0, lz, 0.0)
    dists0 = jnp.full((B, n), jnp.inf, dtype=jnp.float32)

    hn = n // 2
    laneL = lane[:, 0:hn]
    laneR = lane[:, hn:n]

    def body(i, state):
        dists, qx, qy, qz, lx, ly, lz = state
        dx = px - lx
        dy = py - ly
        dz = pz - lz
        d = (dx * dx + dy * dy) + dz * dz
        dists = jnp.minimum(dists, d)
        dL = dists[:, 0:hn]
        dR = dists[:, hn:n]
        m = jnp.maximum(
            jnp.max(dL, axis=1, keepdims=True),
            jnp.max(dR, axis=1, keepdims=True),
        )
        sel = jnp.minimum(
            jnp.min(jnp.where(dL == m, laneL, n), axis=1, keepdims=True),
            jnp.min(jnp.where(dR == m, laneR, n), axis=1, keepdims=True),
        )
        mskL = laneL == sel
        mskR = laneR == sel
        nlx = jnp.sum(jnp.where(mskL, px[:, 0:hn], 0.0), axis=1, keepdims=True) + \
              jnp.sum(jnp.where(mskR, px[:, hn:n], 0.0), axis=1, keepdims=True)
        nly = jnp.sum(jnp.where(mskL, py[:, 0:hn], 0.0), axis=1, keepdims=True) + \
              jnp.sum(jnp.where(mskR, py[:, hn:n], 0.0), axis=1, keepdims=True)
        nlz = jnp.sum(jnp.where(mskL, pz[:, 0:hn], 0.0), axis=1, keepdims=True) + \
              jnp.sum(jnp.where(mskR, pz[:, hn:n], 0.0), axis=1, keepdims=True)
        at_i = lane_s == i
        qx = jnp.where(at_i, nlx, qx)
        qy = jnp.where(at_i, nly, qy)
        qz = jnp.where(at_i, nlz, qz)
        return dists, qx, qy, qz, nlx, nly, nlz

    _, qx, qy, qz, _, _, _ = lax.fori_loop(
        1, n_samples, body, (dists0, qx0, qy0, qz0, lx, ly, lz)
    )
    q_ref[0] = qx
    q_ref[1] = qy
    q_ref[2] = qz


def _fps(pos_coord, n, n_samples):
    return pl.pallas_call(
        functools.partial(_fps_body, n=n, n_samples=n_samples),
        grid=(1,),
        in_specs=[pl.BlockSpec((3, B, n), lambda i: (0, 0, 0))],
        out_specs=pl.BlockSpec((3, B, n_samples), lambda i: (0, 0, 0)),
        out_shape=jax.ShapeDtypeStruct((3, B, n_samples), jnp.float32),
    )(pos_coord)


def _knn_body(pos_cols_ref, q_rows_ref, idx_ref, cnt_ref, work_ref, *, n, q, r2s):
    pxc = pos_cols_ref[0, :, 0:1]
    pyc = pos_cols_ref[0, :, 1:2]
    pzc = pos_cols_ref[0, :, 2:3]
    qx = q_rows_ref[0, 0:1, :]
    qy = q_rows_ref[0, 1:2, :]
    qz = q_rows_ref[0, 2:3, :]
    dx = qx - pxc
    dy = qy - pyc
    dz = qz - pzc
    work_ref[...] = (dx * dx + dy * dy) + dz * dz
    h = n // 2
    subL = lax.broadcasted_iota(jnp.int32, (h, q), 0)
    subR = subL + h
    ksub = lax.broadcasted_iota(jnp.int32, (K, q), 0)

    def body(k, state):
        idxc, c0, c1 = state
        wL = work_ref[0:h, :]
        wR = work_ref[h:n, :]
        mL = jnp.min(wL, axis=0, keepdims=True)
        mR = jnp.min(wR, axis=0, keepdims=True)
        m = jnp.minimum(mL, mR)
        miL = jnp.min(jnp.where(wL == m, subL, n), axis=0, keepdims=True)
        miR = jnp.min(jnp.where(wR == m, subR, n), axis=0, keepdims=True)
        mi = jnp.minimum(miL, miR)
        idxc = jnp.where(ksub == k, mi, idxc)
        c0 = c0 + jnp.where(m <= r2s[0], 1, 0)
        c1 = c1 + jnp.where(m <= r2s[1], 1, 0)
        work_ref[0:h, :] = jnp.where(subL == mi, jnp.inf, wL)
        work_ref[h:n, :] = jnp.where(subR == mi, jnp.inf, wR)
        return idxc, c0, c1

    idxc0 = jnp.zeros((K, q), jnp.int32)
    z = jnp.zeros((1, q), jnp.int32)
    idxc, c0, c1 = lax.fori_loop(0, K, body, (idxc0, z, z))
    idx_ref[0, :, :] = idxc
    cnt_ref[0, 0:1, :] = c0
    cnt_ref[0, 1:2, :] = c1


def _knn(pos_cols, q_rows, n, q, r2s):
    nr = len(r2s)
    return pl.pallas_call(
        functools.partial(_knn_body, n=n, q=q, r2s=r2s),
        grid=(B,),
        in_specs=[
            pl.BlockSpec((1, n, 3), lambda b: (b, 0, 0)),
            pl.BlockSpec((1, 3, q), lambda b: (b, 0, 0)),
        ],
        out_specs=[
            pl.BlockSpec((1, K, q), lambda b: (b, 0, 0)),
            pl.BlockSpec((1, nr, q), lambda b: (b, 0, 0)),
        ],
        out_shape=[
            jax.ShapeDtypeStruct((B, K, q), jnp.int32),
            jax.ShapeDtypeStruct((B, nr, q), jnp.int32),
        ],
        scratch_shapes=[pltpu.VMEM((n, q), jnp.float32)],
        compiler_params=pltpu.CompilerParams(dimension_semantics=("parallel",)),
    )(pos_cols, q_rows)


def _a1_body(pos_cols_ref, col_cols_ref, w_ref, b_ref, a_ref, *, n, f):
    acc = jnp.broadcast_to(b_ref[0:1, :], (n, f))
    for c in range(3):
        acc = acc + col_cols_ref[0, :, c : c + 1] * w_ref[c : c + 1, :]
    for c in range(3):
        acc = acc + pos_cols_ref[0, :, c : c + 1] * w_ref[3 + c : 4 + c, :]
    a_ref[0, :, :] = acc


def _a1(pos_cols, color_cols, w1, b1):
    f = w1.shape[1]
    return pl.pallas_call(
        functools.partial(_a1_body, n=N, f=f),
        grid=(B,),
        in_specs=[
            pl.BlockSpec((1, N, 3), lambda b: (b, 0, 0)),
            pl.BlockSpec((1, N, 3), lambda b: (b, 0, 0)),
            pl.BlockSpec((6, f), lambda b: (0, 0)),
            pl.BlockSpec((1, f), lambda b: (0, 0)),
        ],
        out_specs=pl.BlockSpec((1, N, f), lambda b: (b, 0, 0)),
        out_shape=jax.ShapeDtypeStruct((B, N, f), jnp.float32),
        compiler_params=pltpu.CompilerParams(dimension_semantics=("parallel",)),
    )(pos_cols, color_cols, w1, b1)


def _a2_body(x_ref, pos_cols_ref, wx_ref, wp_ref, b_ref, a_ref, *, n, f):
    acc = jnp.dot(x_ref[0], wx_ref[...], preferred_element_type=jnp.float32)
    acc = acc + b_ref[0:1, :]
    for c in range(3):
        acc = acc + pos_cols_ref[0, :, c : c + 1] * wp_ref[c : c + 1, :]
    a_ref[0, :, :] = acc


def _a2(x1, pos_cols, wx, wp, b):
    n, fin = x1.shape[1], x1.shape[2]
    f = wx.shape[1]
    return pl.pallas_call(
        functools.partial(_a2_body, n=n, f=f),
        grid=(B,),
        in_specs=[
            pl.BlockSpec((1, n, fin), lambda b: (b, 0, 0)),
            pl.BlockSpec((1, n, 3), lambda b: (b, 0, 0)),
            pl.BlockSpec((fin, f), lambda b: (0, 0)),
            pl.BlockSpec((3, f), lambda b: (0, 0)),
            pl.BlockSpec((1, f), lambda b: (0, 0)),
        ],
        out_specs=pl.BlockSpec((1, n, f), lambda b: (b, 0, 0)),
        out_shape=jax.ShapeDtypeStruct((B, n, f), jnp.float32),
        compiler_params=pltpu.CompilerParams(dimension_semantics=("parallel",)),
    )(x1, pos_cols, wx, wp, b)


def _sc_gather(table, idx, chunk):
    info = plsc.get_sparse_core_info()
    nc, ns = info.num_cores, info.num_subcores
    nw = nc * ns
    rows, d = idx.shape[0], table.shape[1]
    b_per_w = rows // nw
    n_chunks = b_per_w // chunk
    mesh = plsc.VectorSubcoreMesh(core_axis_name="c", subcore_axis_name="s")

    @functools.partial(
        pl.kernel,
        mesh=mesh,
        out_type=jax.ShapeDtypeStruct((rows, d), jnp.float32),
        scratch_types=[
            pltpu.VMEM((chunk,), jnp.int32),
            pltpu.VMEM((chunk, d), jnp.float32),
            pltpu.SemaphoreType.DMA,
        ],
    )
    def k(table_hbm, idx_hbm, out_hbm, idx_v, rows_v, sem):
        wid = lax.axis_index("s") * nc + lax.axis_index("c")
        base = wid * b_per_w

        @pl.loop(0, n_chunks)
        def _chunk(c):
            off = base + c * chunk
            pltpu.sync_copy(idx_hbm.at[pl.ds(off, chunk)], idx_v)
            pltpu.async_copy(table_hbm.at[idx_v], rows_v, sem).wait()
            pltpu.sync_copy(rows_v, out_hbm.at[pl.ds(off, chunk)])

    return k(table, idx)


def _conv_body(ag_ref, q_cols_ref, cnt_cols_ref, w1p_ref, s1_ref, t1_ref,
               w2_ref, b2_ref, s2_ref, t2_ref, out_ref, *, q, f1, f2, gw):
    cq = q_cols_ref[0, :, 0:1] * w1p_ref[0:1, :]
    cq = cq + q_cols_ref[0, :, 1:2] * w1p_ref[1:2, :]
    cq = cq + q_cols_ref[0, :, 2:3] * w1p_ref[2:3, :]
    cnt0 = cnt_cols_ref[0, :, 0:1]
    cnt1 = cnt_cols_ref[0, :, 1:2]

    w2 = w2_ref[...]

    def half(k, acc0, acc1):
        h1 = jax.nn.relu(ag_ref[0, k][:, 0:f1] - cq)
        h1 = h1 * s1_ref[0:1, :] + t1_ref[0:1, :]
        h2 = jnp.dot(h1, w2, preferred_element_type=jnp.float32)
        h2 = jax.nn.relu(h2 + b2_ref[0:1, :])
        h2 = h2 * s2_ref[0:1, :] + t2_ref[0:1, :]
        acc0 = jnp.where(k < cnt0, jnp.maximum(acc0, h2), acc0)
        acc1 = jnp.where(k < cnt1, jnp.maximum(acc1, h2), acc1)
        return acc0, acc1

    def body(k, state):
        a0a, a1a, a0b, a1b = state
        a0a, a1a = half(k, a0a, a1a)
        a0b, a1b = half(k + K // 2, a0b, a1b)
        return a0a, a1a, a0b, a1b

    neg = jnp.full((q, f2), NEG_INF, dtype=jnp.float32)
    a0a, a1a, a0b, a1b = lax.fori_loop(0, K // 2, body, (neg, neg, neg, neg))
    acc0 = jnp.maximum(a0a, a0b)
    acc1 = jnp.maximum(a1a, a1b)
    out_ref[0, :, 0:f2] = jnp.where(acc0 > NEG_INF, acc0, 0.0)
    out_ref[0, :, f2 : 2 * f2] = jnp.where(acc1 > NEG_INF, acc1, 0.0)


def _conv(ag, q_cols, cnt_cols, w1p, s1, t1, w2, b2, s2, t2, q, qb=None):
    f1, f2 = w2.shape
    gw = ag.shape[3]
    qb = q if qb is None else qb
    return pl.pallas_call(
        functools.partial(_conv_body, q=qb, f1=f1, f2=f2, gw=gw),
        grid=(B, q // qb),
        in_specs=[
            pl.BlockSpec((1, K, qb, gw), lambda b, i: (b, 0, i, 0)),
            pl.BlockSpec((1, qb, 3), lambda b, i: (b, i, 0)),
            pl.BlockSpec((1, qb, 2), lambda b, i: (b, i, 0)),
            pl.BlockSpec((3, f1), lambda b, i: (0, 0)),
            pl.BlockSpec((1, f1), lambda b, i: (0, 0)),
            pl.BlockSpec((1, f1), lambda b, i: (0, 0)),
            pl.BlockSpec((f1, f2), lambda b, i: (0, 0)),
            pl.BlockSpec((1, f2), lambda b, i: (0, 0)),
            pl.BlockSpec((1, f2), lambda b, i: (0, 0)),
            pl.BlockSpec((1, f2), lambda b, i: (0, 0)),
        ],
        out_specs=pl.BlockSpec((1, qb, 2 * f2), lambda b, i: (b, i, 0)),
        out_shape=jax.ShapeDtypeStruct((B, q, 2 * f2), jnp.float32),
        compiler_params=pltpu.CompilerParams(
            dimension_semantics=("parallel", "arbitrary")
        ),
    )(ag, q_cols, cnt_cols, w1p, s1, t1, w2, b2, s2, t2)


def _head_a_body(x_ref, pos_cols_ref, wx_ref, wp_ref, b_ref, s_ref, t_ref,
                 out_ref, *, n, f):
    h = jnp.dot(x_ref[0], wx_ref[...], preferred_element_type=jnp.float32)
    h = h + b_ref[0:1, :]
    for c in range(3):
        h = h + pos_cols_ref[0, :, c : c + 1] * wp_ref[c : c + 1, :]
    h = jax.nn.relu(h)
    h = h * s_ref[0:1, :] + t_ref[0:1, :]
    out_ref[0, :, :] = jnp.max(h, axis=0, keepdims=True)


def _head_a(x2, pos_cols, wx, wp, b, s, t):
    n, fin = x2.shape[1], x2.shape[2]
    f = wx.shape[1]
    return pl.pallas_call(
        functools.partial(_head_a_body, n=n, f=f),
        grid=(B,),
        in_specs=[
            pl.BlockSpec((1, n, fin), lambda b: (b, 0, 0)),
            pl.BlockSpec((1, n, 3), lambda b: (b, 0, 0)),
            pl.BlockSpec((fin, f), lambda b: (0, 0)),
            pl.BlockSpec((3, f), lambda b: (0, 0)),
            pl.BlockSpec((1, f), lambda b: (0, 0)),
            pl.BlockSpec((1, f), lambda b: (0, 0)),
            pl.BlockSpec((1, f), lambda b: (0, 0)),
        ],
        out_specs=pl.BlockSpec((1, 1, f), lambda b: (b, 0, 0)),
        out_shape=jax.ShapeDtypeStruct((B, 1, f), jnp.float32),
        compiler_params=pltpu.CompilerParams(dimension_semantics=("parallel",)),
    )(x2, pos_cols, wx, wp, b, s, t)


def _head_b_body(x_ref, w1_ref, b1_ref, w2_ref, b2_ref, out_ref):
    h = jnp.dot(x_ref[0], w1_ref[...], preferred_element_type=jnp.float32)
    h = jax.nn.relu(h + b1_ref[0:1, :])
    o = jnp.dot(h, w2_ref[...], preferred_element_type=jnp.float32)
    o = o + b2_ref[0:1, :]
    nrm = jnp.sqrt(jnp.sum(o * o, axis=1, keepdims=True))
    out_ref[0, :, :] = o / nrm


def _head_b(x3, w1, b1, w2, b2):
    f1, f2 = w1.shape[1], w2.shape[1]
    return pl.pallas_call(
        _head_b_body,
        grid=(1,),
        in_specs=[
            pl.BlockSpec((1, B, x3.shape[2]), lambda i: (0, 0, 0)),
            pl.BlockSpec(w1.shape, lambda i: (0, 0)),
            pl.BlockSpec((1, f1), lambda i: (0, 0)),
            pl.BlockSpec(w2.shape, lambda i: (0, 0)),
            pl.BlockSpec((1, f2), lambda i: (0, 0)),
        ],
        out_specs=pl.BlockSpec((1, B, f2), lambda i: (0, 0, 0)),
        out_shape=jax.ShapeDtypeStruct((1, B, f2), jnp.float32),
    )(x3, w1, b1, w2, b2)


def _bn_fold(layer):
    w, b, g, be = layer
    s = g / jnp.sqrt(1.0 + EPS)
    return w, b.reshape(1, -1), s.reshape(1, -1), be.reshape(1, -1)


def kernel(xyz, color, params):
    sa1 = [_bn_fold(l) for l in params["sa1"]]
    sa2 = [_bn_fold(l) for l in params["sa2"]]
    sa3 = [_bn_fold(l) for l in params["sa3"]]
    w1_1, b1_1, s1_1, t1_1 = sa1[0]
    w2_1, b2_1, s2_1, t2_1 = sa1[1]
    w1_2, b1_2, s1_2, t1_2 = sa2[0]
    w2_2, b2_2, s2_2, t2_2 = sa2[1]
    w3, b3, s3, t3 = sa3[0]
    wl1, bl1 = params["lin1"]
    wl2, bl2 = params["lin2"]

    pos_coord = jnp.transpose(xyz, (2, 0, 1))
    pos_cols = xyz
    color_cols = color

    q1_coord = _fps(pos_coord, N, N1)
    q1_rows = jnp.swapaxes(q1_coord, 0, 1)
    q1_cols = jnp.transpose(q1_coord, (1, 2, 0))
    nbr1, cnt1 = _knn(pos_cols, q1_rows, N, N1, (0.2 * 0.2, 0.1 * 0.1))
    a1 = _a1(pos_cols, color_cols, w1_1, b1_1)
    a1p = jnp.pad(a1.reshape(B * N, 64), ((0, 0), (0, 64)))
    offs1 = (jnp.arange(B, dtype=jnp.int32) * N).reshape(B, 1, 1)
    idx1 = (nbr1 + offs1).reshape(B * K * N1)
    g1 = _sc_gather(a1p, idx1, 512)
    g1 = g1.reshape(B, K, N1, 128)
    cnt1_cols = jnp.swapaxes(cnt1, 1, 2)
    x1 = _conv(g1, q1_cols, cnt1_cols, w1_1[3:6], s1_1, t1_1,
               w2_1, b2_1, s2_1, t2_1, N1, qb=512)

    q2_coord = _fps(q1_coord, N1, N2)
    q2_rows = jnp.swapaxes(q2_coord, 0, 1)
    q2_cols = jnp.transpose(q2_coord, (1, 2, 0))
    nbr2, cnt2 = _knn(q1_cols, q2_rows, N1, N2, (0.35 * 0.35, 0.5 * 0.5))
    a2 = _a2(x1, q1_cols, w1_2[:128], w1_2[128:131], b1_2)
    offs2 = (jnp.arange(B, dtype=jnp.int32) * N1).reshape(B, 1, 1)
    idx2 = (nbr2 + offs2).reshape(B * K * N2)
    g2 = _sc_gather(a2.reshape(B * N1, 128), idx2, 656)
    g2 = g2.reshape(B, K, N2, 128)
    cnt2_cols = jnp.swapaxes(cnt2, 1, 2)
    x2 = _conv(g2, q2_cols, cnt2_cols, w1_2[128:131], s1_2, t1_2,
               w2_2, b2_2, s2_2, t2_2, N2)

    x3 = _head_a(x2, q2_cols, w3[:512], w3[512:515], b3, s3, t3)
    x3 = x3.reshape(1, B, 1024)
    out = _head_b(x3, wl1, bl1.reshape(1, -1), wl2, bl2.reshape(1, -1))
    return out.reshape(B, OUT)
